# baseline (speedup 1.0000x reference)
import dataclasses
import functools

import jax
import jax.numpy as jnp
from jax import lax
from jax.experimental import pallas as pl
from jax.experimental.pallas import tpu as pltpu
from jax.experimental.pallas import tpu_sc as plsc

N = 10000
NP = 10240
E = 320000
F = 128
G = 64
EPS = 1e-5

NC = 2
NS = 16
NW = NC * NS
PER_TILE = E // NW
CH = 40
NCHUNK = PER_TILE // CH
ROWS_PER_TILE = NP // NS

_mesh = plsc.VectorSubcoreMesh(core_axis_name="c", subcore_axis_name="s")

_cp_no_layout = pltpu.CompilerParams()
if "needs_layout_passes" in pltpu.CompilerParams.__dataclass_fields__:
    _cp_no_layout = dataclasses.replace(_cp_no_layout,
                                        needs_layout_passes=False)

_cp_sc_compact = pltpu.CompilerParams(use_tc_tiling_on_sc=False)


@functools.partial(
    pl.kernel,
    out_type=jax.ShapeDtypeStruct((NW * N,), jnp.float32),
    mesh=_mesh,
    scratch_types=[
        pltpu.VMEM((PER_TILE,), jnp.int32),
        pltpu.VMEM((N,), jnp.float32),
    ],
    compiler_params=_cp_no_layout,
)
def _hist(cidx_hbm, out_hbm, cidx_v, cnt):
    ci = lax.axis_index("c")
    si = lax.axis_index("s")
    wid = ci * NS + si
    zeros = jnp.zeros((16,), jnp.float32)

    @pl.loop(0, N, step=16)
    def _(i):
        cnt[pl.ds(i, 16)] = zeros

    pltpu.sync_copy(cidx_hbm.at[pl.ds(wid * PER_TILE, PER_TILE)], cidx_v)
    ones = jnp.ones((16,), jnp.float32)

    @pl.loop(0, PER_TILE, step=16)
    def _(j):
        idx = cidx_v[pl.ds(j, 16)]
        plsc.addupdate_scatter(cnt, [idx], ones)

    pltpu.sync_copy(cnt, out_hbm.at[pl.ds(wid * N, N)])


NB = 5


@functools.partial(
    pl.kernel,
    out_type=jax.ShapeDtypeStruct((NC, NP, F), jnp.float32),
    mesh=_mesh,
    scratch_types=[
        pltpu.VMEM((NCHUNK, CH), jnp.int32),
        pltpu.VMEM((NCHUNK, CH), jnp.int32),
    ] + [pltpu.VMEM((CH, F), jnp.float32) for _ in range(4)] + [
        pltpu.VMEM_SHARED((NP, F), jnp.float32),
        pltpu.SemaphoreType.DMA,
        pltpu.SemaphoreType.DMA,
    ],
    compiler_params=_cp_sc_compact,
)
def _scatter(hp_hbm, ridx_hbm, cidx_hbm, zeros_hbm, out_hbm,
             ridx_v, cidx_v, *rest):
    bufs = rest[:4]
    acc = rest[4]
    gsem = rest[5]
    ssem = rest[6]
    ci = lax.axis_index("c")
    si = lax.axis_index("s")
    wid = ci * NS + si
    row0 = si * ROWS_PER_TILE
    pltpu.sync_copy(ridx_hbm.at[wid], ridx_v)
    pltpu.sync_copy(cidx_hbm.at[wid], cidx_v)
    pltpu.sync_copy(zeros_hbm.at[pl.ds(row0, ROWS_PER_TILE)],
                    acc.at[pl.ds(row0, ROWS_PER_TILE)])
    plsc.subcore_barrier()

    def _wait_gather(k, s):
        pltpu.make_async_copy(hp_hbm.at[ridx_v.at[k]], bufs[s], gsem).wait()

    def _wait_scatter(s):
        pltpu.make_async_copy(bufs[s], acc.at[cidx_v.at[0]], ssem).wait()

    def _issue_gather(k, s):
        kc = jnp.minimum(k, NCHUNK - 1)
        pltpu.async_copy(hp_hbm.at[ridx_v.at[kc]], bufs[s], gsem)

    def _issue_scatter(k, s):
        pltpu.async_copy(bufs[s], acc.at[cidx_v.at[k]], ssem, add=True)

    def _body(j, s, ssem_wait):
        _wait_gather(j, s)
        _issue_scatter(j, s)
        if ssem_wait:
            _wait_scatter(s)
        _issue_gather(j + 2, (s + 2) % 4)

    for s in range(2):
        _issue_gather(s, s)
    for j in range(4):
        _body(j, j % 4, j >= 2)

    @pl.loop(4, NCHUNK - 2, step=4)
    def _(j0):
        for b in range(4):
            _body(j0 + b, b, True)

    for j in (NCHUNK - 2, NCHUNK - 1):
        _body(j, j % 4, True)
    for s in range(2):
        _wait_gather(0, s)
        _wait_scatter(s)

    plsc.subcore_barrier()
    pltpu.sync_copy(acc.at[pl.ds(row0, ROWS_PER_TILE)],
                    out_hbm.at[ci, pl.ds(row0, ROWS_PER_TILE)])


PR = NP // NW
PCH = 16
PNCH = PR // PCH
PG = 72


@functools.partial(
    pl.kernel,
    out_type=jax.ShapeDtypeStruct((NW, PG, F), jnp.float32),
    mesh=_mesh,
    scratch_types=[
        pltpu.VMEM((PR,), jnp.int32),
        pltpu.VMEM((PG, F), jnp.float32),
        pltpu.VMEM((PCH, F), jnp.float32),
        pltpu.VMEM((PCH, F), jnp.float32),
        pltpu.SemaphoreType.DMA,
    ],
    compiler_params=_cp_no_layout,
)
def _pool(a_hbm, bid_hbm, out_hbm, bid_v, pool_v, buf0, buf1, psem):
    ci = lax.axis_index("c")
    si = lax.axis_index("s")
    wid = ci * NS + si
    row0 = wid * PR
    pltpu.sync_copy(bid_hbm.at[pl.ds(row0, PR)], bid_v)
    ninf = jnp.full((16,), -jnp.inf, dtype=jnp.float32)

    @pl.loop(0, PG)
    def _(i):
        for j in range(F // 16):
            pool_v[i, pl.ds(j * 16, 16)] = ninf

    bufs = (buf0, buf1)
    for b in range(2):
        pltpu.async_copy(a_hbm.at[pl.ds(row0 + b * PCH, PCH)], bufs[b], psem)

    @pl.loop(0, PNCH, step=2)
    def _(c0):
        for b in range(2):
            c = c0 + b
            buf = bufs[b]
            pltpu.make_async_copy(a_hbm.at[pl.ds(row0, PCH)],
                                  buf, psem).wait()
            bids = bid_v[pl.ds(c * PCH, PCH)]
            for i in range(PCH):
                bid = bids[i]
                for j in range(F // 16):
                    sl = pl.ds(j * 16, 16)
                    pool_v[bid, sl] = jnp.maximum(pool_v[bid, sl],
                                                  buf[i, sl])
            cn = jnp.minimum(c + 2, PNCH - 1)
            pltpu.async_copy(a_hbm.at[pl.ds(row0 + cn * PCH, PCH)],
                             buf, psem)

    for b in range(2):
        pltpu.make_async_copy(a_hbm.at[pl.ds(row0, PCH)], bufs[b],
                              psem).wait()

    pltpu.sync_copy(pool_v, out_hbm.at[wid])


def _scale_body(x_ref, w_ref, cnt_ref, hp_ref, dinv_ref):
    deg = jnp.sum(cnt_ref[...], axis=0)[:, None] + 2.0
    dinv = lax.rsqrt(deg)
    dinv_ref[...] = dinv
    h = jnp.dot(x_ref[...], w_ref[...], preferred_element_type=jnp.float32)
    hp_ref[...] = h * dinv


def _bn_lrelu(t, g, be):
    m = jnp.mean(t, axis=0, keepdims=True)
    d = t - m
    v = jnp.mean(d * d, axis=0, keepdims=True)
    bn = d * lax.rsqrt(v + EPS) * g + be
    return jnp.where(bn >= 0, bn, 0.1 * bn)


def _mid_body(acc_ref, hp_ref, dinv_ref, g_ref, be_ref, w_ref, o_ref):
    dinv = dinv_ref[...]
    t = dinv * (acc_ref[0, 0:N] + acc_ref[1, 0:N] + 2.0 * hp_ref[...])
    a = _bn_lrelu(t, g_ref[...], be_ref[...])
    o_ref[...] = jnp.dot(a, w_ref[...],
                         preferred_element_type=jnp.float32) * dinv


def _bn2_body(acc_ref, hp_ref, dinv_ref, g_ref, be_ref, o_ref):
    dinv = dinv_ref[...]
    t = dinv * (acc_ref[0, 0:N] + acc_ref[1, 0:N] + 2.0 * hp_ref[...])
    o_ref[0:N, :] = _bn_lrelu(t, g_ref[...], be_ref[...])
    o_ref[N:NP, :] = jnp.zeros((NP - N, F), jnp.float32)


def _proj_body(pools_ref, wf_ref, bf_ref, o_ref):
    pooled = jnp.max(pools_ref[:, 0:G, :], axis=0)
    o_ref[...] = jnp.dot(pooled, wf_ref[...],
                         preferred_element_type=jnp.float32) + bf_ref[...]


def _tc(body, out_shape, *args):
    return pl.pallas_call(
        body, out_shape=jax.ShapeDtypeStruct(out_shape, jnp.float32))(*args)


def kernel(x, edge_index, batch, W1, b1, g1, be1, W2, b2, g2, be2, Wf, bf):
    r = edge_index[0]
    c = edge_index[1]
    r3 = r.reshape(NW, NCHUNK, CH)
    c3 = c.reshape(NW, NCHUNK, CH)
    zeros_nf = jnp.zeros((NP, F), jnp.float32)

    cnt = _hist(c).reshape(NW, N)
    hp1, dinv = pl.pallas_call(
        _scale_body,
        out_shape=(jax.ShapeDtypeStruct((N, F), jnp.float32),
                   jax.ShapeDtypeStruct((N, 1), jnp.float32)),
    )(x, W1, cnt)

    acc1 = _scatter(hp1, r3, c3, zeros_nf)
    hp2 = _tc(_mid_body, (N, F), acc1, hp1, dinv,
              g1.reshape(1, F), be1.reshape(1, F), W2)

    acc2 = _scatter(hp2, r3, c3, zeros_nf)
    a2p = _tc(_bn2_body, (NP, F), acc2, hp2, dinv,
              g2.reshape(1, F), be2.reshape(1, F))
    bid = jnp.concatenate([batch, jnp.full((NP - N,), G, jnp.int32)])
    pools = _pool(a2p, bid)
    out = _tc(_proj_body, (G, F), pools, Wf, bf.reshape(1, F))
    return out


# speedup vs baseline: 1.2672x; 1.2672x over previous
import dataclasses
import functools

import jax
import jax.numpy as jnp
from jax import lax
from jax.experimental import pallas as pl
from jax.experimental.pallas import tpu as pltpu
from jax.experimental.pallas import tpu_sc as plsc

N = 10000
NP = 10240
E = 320000
F = 128
G = 64
EPS = 1e-5

NC = 2
NS = 16
NW = NC * NS
PER_TILE = E // NW
CH = 40
NCHUNK = PER_TILE // CH
ROWS_PER_TILE = NP // NS

_mesh = plsc.VectorSubcoreMesh(core_axis_name="c", subcore_axis_name="s")

_cp_no_layout = pltpu.CompilerParams()
if "needs_layout_passes" in pltpu.CompilerParams.__dataclass_fields__:
    _cp_no_layout = dataclasses.replace(_cp_no_layout,
                                        needs_layout_passes=False)

_cp_sc_compact = pltpu.CompilerParams(use_tc_tiling_on_sc=False)


@functools.partial(
    pl.kernel,
    out_type=jax.ShapeDtypeStruct((NW * N,), jnp.float32),
    mesh=_mesh,
    scratch_types=[
        pltpu.VMEM((PER_TILE,), jnp.int32),
        pltpu.VMEM((N,), jnp.float32),
    ],
    compiler_params=_cp_no_layout,
)
def _hist(cidx_hbm, out_hbm, cidx_v, cnt):
    ci = lax.axis_index("c")
    si = lax.axis_index("s")
    wid = ci * NS + si
    zeros = jnp.zeros((16,), jnp.float32)

    @pl.loop(0, N, step=16)
    def _(i):
        cnt[pl.ds(i, 16)] = zeros

    pltpu.sync_copy(cidx_hbm.at[pl.ds(wid * PER_TILE, PER_TILE)], cidx_v)
    ones = jnp.ones((16,), jnp.float32)

    @pl.loop(0, PER_TILE, step=16)
    def _(j):
        idx = cidx_v[pl.ds(j, 16)]
        plsc.addupdate_scatter(cnt, [idx], ones)

    pltpu.sync_copy(cnt, out_hbm.at[pl.ds(wid * N, N)])


NB = 5


@functools.partial(
    pl.kernel,
    out_type=jax.ShapeDtypeStruct((NC, NP, F), jnp.float32),
    mesh=_mesh,
    scratch_types=[
        pltpu.VMEM((NCHUNK, CH), jnp.int32),
        pltpu.VMEM((NCHUNK, CH), jnp.int32),
    ] + [pltpu.VMEM((CH, F), jnp.float32) for _ in range(NB)] + [
        pltpu.VMEM_SHARED((NP, F), jnp.float32),
        pltpu.SemaphoreType.DMA,
    ],
    compiler_params=_cp_sc_compact,
)
def _scatter(hp_hbm, ridx_hbm, cidx_hbm, zeros_hbm, out_hbm,
             ridx_v, cidx_v, *rest):
    bufs = rest[:NB]
    acc = rest[NB]
    gsem = rest[NB + 1]
    ci = lax.axis_index("c")
    si = lax.axis_index("s")
    wid = ci * NS + si
    row0 = si * ROWS_PER_TILE
    pltpu.sync_copy(ridx_hbm.at[wid], ridx_v)
    pltpu.sync_copy(cidx_hbm.at[wid], cidx_v)
    pltpu.sync_copy(zeros_hbm.at[pl.ds(row0, ROWS_PER_TILE)],
                    acc.at[pl.ds(row0, ROWS_PER_TILE)])
    plsc.subcore_barrier()

    for b in range(NB):
        pltpu.async_copy(hp_hbm.at[ridx_v.at[b]], bufs[b], gsem)

    @pl.loop(0, NCHUNK, step=NB)
    def _(k0):
        for b in range(NB):
            k = k0 + b
            pltpu.make_async_copy(hp_hbm.at[ridx_v.at[k]],
                                  bufs[b], gsem).wait()
            pltpu.sync_copy(bufs[b], acc.at[cidx_v.at[k]], add=True)
            kn = jnp.minimum(k + NB, NCHUNK - 1)
            pltpu.async_copy(hp_hbm.at[ridx_v.at[kn]], bufs[b], gsem)

    for b in range(NB):
        pltpu.make_async_copy(hp_hbm.at[ridx_v.at[0]], bufs[b], gsem).wait()

    plsc.subcore_barrier()
    pltpu.sync_copy(acc.at[pl.ds(row0, ROWS_PER_TILE)],
                    out_hbm.at[ci, pl.ds(row0, ROWS_PER_TILE)])


PR = NP // NW
PCH = 16
PNCH = PR // PCH
PG = 72


@functools.partial(
    pl.kernel,
    out_type=jax.ShapeDtypeStruct((NW, PG, F), jnp.float32),
    mesh=_mesh,
    scratch_types=[
        pltpu.VMEM((PR,), jnp.int32),
        pltpu.VMEM((PG, F), jnp.float32),
        pltpu.VMEM((PCH, F), jnp.float32),
        pltpu.VMEM((PCH, F), jnp.float32),
        pltpu.SemaphoreType.DMA,
    ],
    compiler_params=_cp_no_layout,
)
def _pool(a_hbm, bid_hbm, out_hbm, bid_v, pool_v, buf0, buf1, psem):
    ci = lax.axis_index("c")
    si = lax.axis_index("s")
    wid = ci * NS + si
    row0 = wid * PR
    pltpu.sync_copy(bid_hbm.at[pl.ds(row0, PR)], bid_v)
    ninf = jnp.full((16,), -jnp.inf, dtype=jnp.float32)

    @pl.loop(0, PG)
    def _(i):
        for j in range(F // 16):
            pool_v[i, pl.ds(j * 16, 16)] = ninf

    bufs = (buf0, buf1)
    for b in range(2):
        pltpu.async_copy(a_hbm.at[pl.ds(row0 + b * PCH, PCH)], bufs[b], psem)

    @pl.loop(0, PNCH, step=2)
    def _(c0):
        for b in range(2):
            c = c0 + b
            buf = bufs[b]
            pltpu.make_async_copy(a_hbm.at[pl.ds(row0, PCH)],
                                  buf, psem).wait()
            bids = bid_v[pl.ds(c * PCH, PCH)]
            for i in range(PCH):
                bid = bids[i]
                for j in range(F // 16):
                    sl = pl.ds(j * 16, 16)
                    pool_v[bid, sl] = jnp.maximum(pool_v[bid, sl],
                                                  buf[i, sl])
            cn = jnp.minimum(c + 2, PNCH - 1)
            pltpu.async_copy(a_hbm.at[pl.ds(row0 + cn * PCH, PCH)],
                             buf, psem)

    for b in range(2):
        pltpu.make_async_copy(a_hbm.at[pl.ds(row0, PCH)], bufs[b],
                              psem).wait()

    pltpu.sync_copy(pool_v, out_hbm.at[wid])


def _scale_body(x_ref, w_ref, cnt_ref, hp_ref, dinv_ref):
    deg = jnp.sum(cnt_ref[...], axis=0)[:, None] + 2.0
    dinv = lax.rsqrt(deg)
    dinv_ref[...] = dinv
    h = jnp.dot(x_ref[...], w_ref[...], preferred_element_type=jnp.float32)
    hp_ref[...] = h * dinv


def _bn_lrelu(t, g, be):
    m = jnp.mean(t, axis=0, keepdims=True)
    d = t - m
    v = jnp.mean(d * d, axis=0, keepdims=True)
    bn = d * lax.rsqrt(v + EPS) * g + be
    return jnp.where(bn >= 0, bn, 0.1 * bn)


def _mid_body(acc_ref, hp_ref, dinv_ref, g_ref, be_ref, w_ref, o_ref):
    dinv = dinv_ref[...]
    t = dinv * (acc_ref[0, 0:N] + acc_ref[1, 0:N] + 2.0 * hp_ref[...])
    a = _bn_lrelu(t, g_ref[...], be_ref[...])
    o_ref[...] = jnp.dot(a, w_ref[...],
                         preferred_element_type=jnp.float32) * dinv


def _bn2_body(acc_ref, hp_ref, dinv_ref, g_ref, be_ref, o_ref):
    dinv = dinv_ref[...]
    t = dinv * (acc_ref[0, 0:N] + acc_ref[1, 0:N] + 2.0 * hp_ref[...])
    o_ref[0:N, :] = _bn_lrelu(t, g_ref[...], be_ref[...])
    o_ref[N:NP, :] = jnp.zeros((NP - N, F), jnp.float32)


def _proj_body(pools_ref, wf_ref, bf_ref, o_ref):
    pooled = jnp.max(pools_ref[:, 0:G, :], axis=0)
    o_ref[...] = jnp.dot(pooled, wf_ref[...],
                         preferred_element_type=jnp.float32) + bf_ref[...]


def _tc(body, out_shape, *args):
    return pl.pallas_call(
        body, out_shape=jax.ShapeDtypeStruct(out_shape, jnp.float32))(*args)


def kernel(x, edge_index, batch, W1, b1, g1, be1, W2, b2, g2, be2, Wf, bf):
    r = edge_index[0]
    c = edge_index[1]
    r3 = r.reshape(NW, NCHUNK, CH)
    c3 = c.reshape(NW, NCHUNK, CH)
    zeros_nf = jnp.zeros((NP, F), jnp.float32)

    cnt = _hist(c).reshape(NW, N)
    hp1, dinv = pl.pallas_call(
        _scale_body,
        out_shape=(jax.ShapeDtypeStruct((N, F), jnp.float32),
                   jax.ShapeDtypeStruct((N, 1), jnp.float32)),
    )(x, W1, cnt)

    acc1 = _scatter(hp1, r3, c3, zeros_nf)
    hp2 = _tc(_mid_body, (N, F), acc1, hp1, dinv,
              g1.reshape(1, F), be1.reshape(1, F), W2)

    acc2 = _scatter(hp2, r3, c3, zeros_nf)
    a2p = _tc(_bn2_body, (NP, F), acc2, hp2, dinv,
              g2.reshape(1, F), be2.reshape(1, F))
    bid = jnp.concatenate([batch, jnp.full((NP - N,), G, jnp.int32)])
    pools = _pool(a2p, bid)
    out = _tc(_proj_body, (G, F), pools, Wf, bf.reshape(1, F))
    return out


# speedup vs baseline: 1.2677x; 1.0004x over previous
import dataclasses
import functools

import jax
import jax.numpy as jnp
from jax import lax
from jax.experimental import pallas as pl
from jax.experimental.pallas import tpu as pltpu
from jax.experimental.pallas import tpu_sc as plsc

N = 10000
NP = 10240
E = 320000
F = 128
G = 64
EPS = 1e-5

NC = 2
NS = 16
NW = NC * NS
PER_TILE = E // NW
CH = 40
NCHUNK = PER_TILE // CH
ROWS_PER_TILE = NP // NS

_mesh = plsc.VectorSubcoreMesh(core_axis_name="c", subcore_axis_name="s")

_cp_no_layout = pltpu.CompilerParams()
if "needs_layout_passes" in pltpu.CompilerParams.__dataclass_fields__:
    _cp_no_layout = dataclasses.replace(_cp_no_layout,
                                        needs_layout_passes=False)

_cp_sc_compact = pltpu.CompilerParams(use_tc_tiling_on_sc=False)


@functools.partial(
    pl.kernel,
    out_type=jax.ShapeDtypeStruct((NW * N,), jnp.float32),
    mesh=_mesh,
    scratch_types=[
        pltpu.VMEM((PER_TILE,), jnp.int32),
        pltpu.VMEM((N,), jnp.float32),
    ],
    compiler_params=_cp_no_layout,
)
def _hist(cidx_hbm, out_hbm, cidx_v, cnt):
    ci = lax.axis_index("c")
    si = lax.axis_index("s")
    wid = ci * NS + si
    zeros = jnp.zeros((16,), jnp.float32)

    @pl.loop(0, N, step=16)
    def _(i):
        cnt[pl.ds(i, 16)] = zeros

    pltpu.sync_copy(cidx_hbm.at[pl.ds(wid * PER_TILE, PER_TILE)], cidx_v)
    ones = jnp.ones((16,), jnp.float32)

    @pl.loop(0, PER_TILE, step=16)
    def _(j):
        idx = cidx_v[pl.ds(j, 16)]
        plsc.addupdate_scatter(cnt, [idx], ones)

    pltpu.sync_copy(cnt, out_hbm.at[pl.ds(wid * N, N)])


NB = 5


@functools.partial(
    pl.kernel,
    out_type=jax.ShapeDtypeStruct((NC, NP, F), jnp.float32),
    mesh=_mesh,
    scratch_types=[
        pltpu.VMEM((PER_TILE,), jnp.int32),
        pltpu.VMEM((NCHUNK, CH), jnp.int32),
    ] + [pltpu.VMEM((CH, F), jnp.float32) for _ in range(NB)] + [
        pltpu.VMEM_SHARED((NP, F), jnp.float32),
        pltpu.SemaphoreType.DMA,
    ],
    compiler_params=_cp_sc_compact,
)
def _scatter(hp_hbm, ridx_hbm, cidx_hbm, zeros_hbm, out_hbm,
             ridx_v, cidx_v, *rest):
    bufs = rest[:NB]
    acc = rest[NB]
    gsem = rest[NB + 1]
    ci = lax.axis_index("c")
    si = lax.axis_index("s")
    wid = ci * NS + si
    row0 = si * ROWS_PER_TILE
    pltpu.sync_copy(ridx_hbm.at[pl.ds(wid * PER_TILE, PER_TILE)], ridx_v)
    pltpu.sync_copy(cidx_hbm.at[wid], cidx_v)
    pltpu.sync_copy(zeros_hbm.at[pl.ds(row0, ROWS_PER_TILE)],
                    acc.at[pl.ds(row0, ROWS_PER_TILE)])
    plsc.subcore_barrier()

    def _ridx(k):
        return ridx_v.at[pl.ds(k * CH, CH)]

    for b in range(NB):
        pltpu.async_copy(hp_hbm.at[_ridx(b)], bufs[b], gsem)

    @pl.loop(0, NCHUNK, step=NB)
    def _(k0):
        for b in range(NB):
            k = k0 + b
            pltpu.make_async_copy(hp_hbm.at[_ridx(k)],
                                  bufs[b], gsem).wait()
            pltpu.sync_copy(bufs[b], acc.at[cidx_v.at[k]], add=True)
            kn = jnp.minimum(k + NB, NCHUNK - 1)
            pltpu.async_copy(hp_hbm.at[_ridx(kn)], bufs[b], gsem)

    for b in range(NB):
        pltpu.make_async_copy(hp_hbm.at[_ridx(0)], bufs[b], gsem).wait()

    plsc.subcore_barrier()
    pltpu.sync_copy(acc.at[pl.ds(row0, ROWS_PER_TILE)],
                    out_hbm.at[ci, pl.ds(row0, ROWS_PER_TILE)])


PR = NP // NW
PCH = 16
PNCH = PR // PCH
PG = 72


@functools.partial(
    pl.kernel,
    out_type=jax.ShapeDtypeStruct((NW, PG, F), jnp.float32),
    mesh=_mesh,
    scratch_types=[
        pltpu.VMEM((PR,), jnp.int32),
        pltpu.VMEM((PG, F), jnp.float32),
        pltpu.VMEM((PCH, F), jnp.float32),
        pltpu.VMEM((PCH, F), jnp.float32),
        pltpu.SemaphoreType.DMA,
    ],
    compiler_params=_cp_no_layout,
)
def _pool(a_hbm, bid_hbm, out_hbm, bid_v, pool_v, buf0, buf1, psem):
    ci = lax.axis_index("c")
    si = lax.axis_index("s")
    wid = ci * NS + si
    row0 = wid * PR
    pltpu.sync_copy(bid_hbm.at[pl.ds(row0, PR)], bid_v)
    ninf = jnp.full((16,), -jnp.inf, dtype=jnp.float32)

    @pl.loop(0, PG)
    def _(i):
        for j in range(F // 16):
            pool_v[i, pl.ds(j * 16, 16)] = ninf

    bufs = (buf0, buf1)
    for b in range(2):
        pltpu.async_copy(a_hbm.at[pl.ds(row0 + b * PCH, PCH)], bufs[b], psem)

    @pl.loop(0, PNCH, step=2)
    def _(c0):
        for b in range(2):
            c = c0 + b
            buf = bufs[b]
            pltpu.make_async_copy(a_hbm.at[pl.ds(row0, PCH)],
                                  buf, psem).wait()
            bids = bid_v[pl.ds(c * PCH, PCH)]
            for i in range(PCH):
                bid = bids[i]
                for j in range(F // 16):
                    sl = pl.ds(j * 16, 16)
                    pool_v[bid, sl] = jnp.maximum(pool_v[bid, sl],
                                                  buf[i, sl])
            cn = jnp.minimum(c + 2, PNCH - 1)
            pltpu.async_copy(a_hbm.at[pl.ds(row0 + cn * PCH, PCH)],
                             buf, psem)

    for b in range(2):
        pltpu.make_async_copy(a_hbm.at[pl.ds(row0, PCH)], bufs[b],
                              psem).wait()

    pltpu.sync_copy(pool_v, out_hbm.at[wid])


def _scale_body(x_ref, w_ref, cnt_ref, hp_ref, dinv_ref):
    deg = jnp.sum(cnt_ref[...], axis=0)[:, None] + 2.0
    dinv = lax.rsqrt(deg)
    dinv_ref[...] = dinv
    h = jnp.dot(x_ref[...], w_ref[...], preferred_element_type=jnp.float32)
    hp_ref[...] = h * dinv


def _bn_lrelu(t, g, be):
    m = jnp.mean(t, axis=0, keepdims=True)
    d = t - m
    v = jnp.mean(d * d, axis=0, keepdims=True)
    bn = d * lax.rsqrt(v + EPS) * g + be
    return jnp.where(bn >= 0, bn, 0.1 * bn)


def _mid_body(acc_ref, hp_ref, dinv_ref, g_ref, be_ref, w_ref, o_ref):
    dinv = dinv_ref[...]
    t = dinv * (acc_ref[0, 0:N] + acc_ref[1, 0:N] + 2.0 * hp_ref[...])
    a = _bn_lrelu(t, g_ref[...], be_ref[...])
    o_ref[...] = jnp.dot(a, w_ref[...],
                         preferred_element_type=jnp.float32) * dinv


def _bn2_body(acc_ref, hp_ref, dinv_ref, g_ref, be_ref, o_ref):
    dinv = dinv_ref[...]
    t = dinv * (acc_ref[0, 0:N] + acc_ref[1, 0:N] + 2.0 * hp_ref[...])
    o_ref[0:N, :] = _bn_lrelu(t, g_ref[...], be_ref[...])
    o_ref[N:NP, :] = jnp.zeros((NP - N, F), jnp.float32)


def _proj_body(pools_ref, wf_ref, bf_ref, o_ref):
    pooled = jnp.max(pools_ref[:, 0:G, :], axis=0)
    o_ref[...] = jnp.dot(pooled, wf_ref[...],
                         preferred_element_type=jnp.float32) + bf_ref[...]


def _tc(body, out_shape, *args):
    return pl.pallas_call(
        body, out_shape=jax.ShapeDtypeStruct(out_shape, jnp.float32))(*args)


def kernel(x, edge_index, batch, W1, b1, g1, be1, W2, b2, g2, be2, Wf, bf):
    r = edge_index[0]
    c = edge_index[1]
    c3 = c.reshape(NW, NCHUNK, CH)
    zeros_nf = jnp.zeros((NP, F), jnp.float32)

    cnt = _hist(c).reshape(NW, N)
    hp1, dinv = pl.pallas_call(
        _scale_body,
        out_shape=(jax.ShapeDtypeStruct((N, F), jnp.float32),
                   jax.ShapeDtypeStruct((N, 1), jnp.float32)),
    )(x, W1, cnt)

    acc1 = _scatter(hp1, r, c3, zeros_nf)
    hp2 = _tc(_mid_body, (N, F), acc1, hp1, dinv,
              g1.reshape(1, F), be1.reshape(1, F), W2)

    acc2 = _scatter(hp2, r, c3, zeros_nf)
    a2p = _tc(_bn2_body, (NP, F), acc2, hp2, dinv,
              g2.reshape(1, F), be2.reshape(1, F))
    bid = jnp.concatenate([batch, jnp.full((NP - N,), G, jnp.int32)])
    pools = _pool(a2p, bid)
    out = _tc(_proj_body, (G, F), pools, Wf, bf.reshape(1, F))
    return out
